# baseline (speedup 1.0000x reference)
import functools

import jax
import jax.numpy as jnp
from jax import lax
from jax.experimental import pallas as pl
from jax.experimental.pallas import tpu as pltpu
from jax.experimental.pallas import tpu_sc as plsc

N_NODES = 10000
N_EDGES = 320000
NODE_DIM = 128
EDGE_DIM = 16
HIDDEN = 64

NUM_CORES = 2
NUM_SUBCORES = 16
NW = NUM_CORES * NUM_SUBCORES
N_HALF = N_EDGES // 2
EPW = N_HALF // NW
CHUNK = 40
N_CHUNKS = EPW // CHUNK


def _proj_body(na_ref, w_ref, b1_ref, p_ref, q_ref):
    pq = jnp.dot(na_ref[...], w_ref[...], preferred_element_type=jnp.float32)
    t = pq + b1_ref[...]
    p_ref[...] = t[:, :HIDDEN]
    q_ref[...] = t[:, HIDDEN:]


def _project_nodes(node_attr, w_sr, bias_row):
    return pl.pallas_call(
        _proj_body,
        out_shape=(
            jax.ShapeDtypeStruct((N_NODES, HIDDEN), jnp.float32),
            jax.ShapeDtypeStruct((N_NODES, HIDDEN), jnp.float32),
        ),
    )(node_attr, w_sr, bias_row)


def _make_gather_add(e0):
    mesh = plsc.VectorSubcoreMesh(core_axis_name="c", subcore_axis_name="s")

    @functools.partial(
        pl.kernel,
        mesh=mesh,
        out_type=jax.ShapeDtypeStruct((N_HALF, 2 * HIDDEN), jnp.float32),
        compiler_params=pltpu.CompilerParams(use_tc_tiling_on_sc=False),
        scratch_types=[
            pltpu.VMEM((EPW,), jnp.int32),
            pltpu.VMEM((EPW,), jnp.int32),
            pltpu.VMEM((CHUNK, HIDDEN), jnp.float32),
            pltpu.VMEM((CHUNK, HIDDEN), jnp.float32),
            pltpu.VMEM((CHUNK, HIDDEN), jnp.float32),
            pltpu.VMEM((CHUNK, HIDDEN), jnp.float32),
            pltpu.VMEM((CHUNK, HIDDEN), jnp.float32),
            pltpu.SemaphoreType.DMA,
            pltpu.SemaphoreType.DMA,
            pltpu.SemaphoreType.DMA,
            pltpu.SemaphoreType.DMA,
        ],
    )
    def gather_add(p_hbm, q_hbm, row_hbm, col_hbm, g_hbm,
                   rows, cols, bufa0, bufb0, bufa1, bufb1, bufo,
                   sa0, sb0, sa1, sb1):
        wid = lax.axis_index("s") * NUM_CORES + lax.axis_index("c")
        wbase = wid * EPW
        pltpu.sync_copy(row_hbm.at[pl.ds(e0 + wbase, EPW)], rows)
        pltpu.sync_copy(col_hbm.at[pl.ds(e0 + wbase, EPW)], cols)

        def fire(ci, ba, bb, sa, sb):
            off = ci * CHUNK
            pltpu.make_async_copy(
                p_hbm.at[rows.at[pl.ds(off, CHUNK)]], ba, sa).start()
            pltpu.make_async_copy(
                q_hbm.at[cols.at[pl.ds(off, CHUNK)]], bb, sb).start()

        def drain(ci, ba, bb, sa, sb):
            off = ci * CHUNK
            pltpu.make_async_copy(
                p_hbm.at[rows.at[pl.ds(off, CHUNK)]], ba, sa).wait()
            pltpu.make_async_copy(
                q_hbm.at[cols.at[pl.ds(off, CHUNK)]], bb, sb).wait()

            def add_body(r, carry):
                for u in range(2):
                    for d in range(HIDDEN // 16):
                        bufo[2 * r + u, pl.ds(d * 16, 16)] = (
                            ba[2 * r + u, pl.ds(d * 16, 16)]
                            + bb[2 * r + u, pl.ds(d * 16, 16)])
                return carry

            lax.fori_loop(0, CHUNK // 2, add_body, 0)
            pltpu.sync_copy(
                bufo, g_hbm.at[pl.ds(wbase + off, CHUNK), pl.ds(0, HIDDEN)])

        fire(0, bufa0, bufb0, sa0, sb0)

        def pair_body(j, carry):
            c0 = 2 * j
            fire(c0 + 1, bufa1, bufb1, sa1, sb1)
            drain(c0, bufa0, bufb0, sa0, sb0)
            fire(c0 + 2, bufa0, bufb0, sa0, sb0)
            drain(c0 + 1, bufa1, bufb1, sa1, sb1)
            return carry

        lax.fori_loop(0, (N_CHUNKS - 1) // 2, pair_body, 0)
        drain(N_CHUNKS - 1, bufa0, bufb0, sa0, sb0)

    return gather_add


BR = 6400


def _mlp_body(eat_ref, g_ref, w1e_ref, w2_ref, b2_ref, w3_ref, b3_ref,
              gm_ref, bt_ref, out_ref):
    h1 = lax.dot_general(eat_ref[...], w1e_ref[...],
                         (((0,), (0,)), ((), ())),
                         preferred_element_type=jnp.float32)
    h1 = jnp.maximum(h1 + g_ref[:, :HIDDEN], 0.0)
    h2 = jnp.dot(h1, w2_ref[...], preferred_element_type=jnp.float32)
    h2 = jnp.maximum(h2 + b2_ref[...], 0.0)
    ot = lax.dot_general(w3_ref[...], h2,
                         (((0,), (1,)), ((), ())),
                         preferred_element_type=jnp.float32) + b3_ref[...]
    mean = jnp.mean(ot, axis=0, keepdims=True)
    c = ot - mean
    var = jnp.mean(c * c, axis=0, keepdims=True)
    out_ref[...] = c * lax.rsqrt(var + 1e-5) * gm_ref[...] + bt_ref[...]


def _mlp(ea_t, g, blk0, w1e, w2, b2_row, w3, b3_col, gamma_col, beta_col):
    n_blocks = N_HALF // BR
    full = lambda i: (0, 0)
    return pl.pallas_call(
        _mlp_body,
        grid=(n_blocks,),
        in_specs=[
            pl.BlockSpec((EDGE_DIM, BR), lambda i: (0, i + blk0)),
            pl.BlockSpec((BR, 2 * HIDDEN), lambda i: (i, 0)),
            pl.BlockSpec((EDGE_DIM, HIDDEN), full),
            pl.BlockSpec((HIDDEN, HIDDEN), full),
            pl.BlockSpec((1, HIDDEN), full),
            pl.BlockSpec((HIDDEN, EDGE_DIM), full),
            pl.BlockSpec((EDGE_DIM, 1), full),
            pl.BlockSpec((EDGE_DIM, 1), full),
            pl.BlockSpec((EDGE_DIM, 1), full),
        ],
        out_specs=pl.BlockSpec((EDGE_DIM, BR), lambda i: (0, i)),
        out_shape=jax.ShapeDtypeStruct((EDGE_DIM, N_HALF), jnp.float32),
    )(ea_t, g, w1e, w2, b2_row, w3, b3_col, gamma_col, beta_col)


def kernel(edge_attr, node_attr, edge_index, W1, b1, W2, b2, W3, b3,
           gamma, beta):
    w1e = W1[:EDGE_DIM]
    w_sr = jnp.concatenate(
        [W1[EDGE_DIM:EDGE_DIM + NODE_DIM], W1[EDGE_DIM + NODE_DIM:]], axis=1)
    bias_row = jnp.concatenate(
        [jnp.zeros((HIDDEN,), jnp.float32), b1]).reshape(1, 2 * HIDDEN)
    p, q = _project_nodes(node_attr, w_sr, bias_row)
    row = edge_index[0]
    col = edge_index[1]
    ea_t = edge_attr.T
    b2r = b2.reshape(1, HIDDEN)
    b3c = b3.reshape(EDGE_DIM, 1)
    gmc = gamma.reshape(EDGE_DIM, 1)
    btc = beta.reshape(EDGE_DIM, 1)
    g0 = _make_gather_add(0)(p, q, row, col)
    g1 = _make_gather_add(N_HALF)(p, q, row, col)
    o0 = _mlp(ea_t, g0, 0, w1e, W2, b2r, W3, b3c, gmc, btc)
    o1 = _mlp(ea_t, g1, N_HALF // BR, w1e, W2, b2r, W3, b3c, gmc, btc)
    return jnp.concatenate([o0, o1], axis=1).T


# speedup vs baseline: 1.2588x; 1.2588x over previous
import functools

import jax
import jax.numpy as jnp
from jax import lax
from jax.experimental import pallas as pl
from jax.experimental.pallas import tpu as pltpu
from jax.experimental.pallas import tpu_sc as plsc

N_NODES = 10000
N_EDGES = 320000
NODE_DIM = 128
EDGE_DIM = 16
HIDDEN = 64

NUM_CORES = 2
NUM_SUBCORES = 16
NW = NUM_CORES * NUM_SUBCORES
N_HALF = N_EDGES // 2
EPW = N_HALF // NW
CHUNK = 200
N_CHUNKS = EPW // CHUNK


def _proj_body(na_ref, w_ref, b1_ref, p_ref, q_ref):
    pq = jnp.dot(na_ref[...], w_ref[...], preferred_element_type=jnp.float32)
    t = pq + b1_ref[...]
    p_ref[...] = t[:, :HIDDEN]
    q_ref[...] = t[:, HIDDEN:]


def _project_nodes(node_attr, w_sr, bias_row):
    return pl.pallas_call(
        _proj_body,
        out_shape=(
            jax.ShapeDtypeStruct((N_NODES, HIDDEN), jnp.float32),
            jax.ShapeDtypeStruct((N_NODES, HIDDEN), jnp.float32),
        ),
    )(node_attr, w_sr, bias_row)


def _make_gather_add(e0):
    mesh = plsc.VectorSubcoreMesh(core_axis_name="c", subcore_axis_name="s")

    @functools.partial(
        pl.kernel,
        mesh=mesh,
        out_type=jax.ShapeDtypeStruct((N_HALF, 2 * HIDDEN), jnp.float32),
        compiler_params=pltpu.CompilerParams(use_tc_tiling_on_sc=False),
        scratch_types=[
            pltpu.VMEM((EPW,), jnp.int32),
            pltpu.VMEM((EPW,), jnp.int32),
            pltpu.VMEM((CHUNK, HIDDEN), jnp.float32),
            pltpu.VMEM((CHUNK, HIDDEN), jnp.float32),
            pltpu.VMEM((CHUNK, HIDDEN), jnp.float32),
            pltpu.VMEM((CHUNK, HIDDEN), jnp.float32),
            pltpu.VMEM((CHUNK, HIDDEN), jnp.float32),
            pltpu.SemaphoreType.DMA,
            pltpu.SemaphoreType.DMA,
            pltpu.SemaphoreType.DMA,
            pltpu.SemaphoreType.DMA,
        ],
    )
    def gather_add(p_hbm, q_hbm, row_hbm, col_hbm, g_hbm,
                   rows, cols, bufa0, bufb0, bufa1, bufb1, bufo,
                   sa0, sb0, sa1, sb1):
        wid = lax.axis_index("s") * NUM_CORES + lax.axis_index("c")
        wbase = wid * EPW
        pltpu.sync_copy(row_hbm.at[pl.ds(e0 + wbase, EPW)], rows)
        pltpu.sync_copy(col_hbm.at[pl.ds(e0 + wbase, EPW)], cols)

        def fire(ci, ba, bb, sa, sb):
            off = ci * CHUNK
            pltpu.make_async_copy(
                p_hbm.at[rows.at[pl.ds(off, CHUNK)]], ba, sa).start()
            pltpu.make_async_copy(
                q_hbm.at[cols.at[pl.ds(off, CHUNK)]], bb, sb).start()

        def drain(ci, ba, bb, sa, sb):
            off = ci * CHUNK
            pltpu.make_async_copy(
                p_hbm.at[rows.at[pl.ds(off, CHUNK)]], ba, sa).wait()
            pltpu.make_async_copy(
                q_hbm.at[cols.at[pl.ds(off, CHUNK)]], bb, sb).wait()

            def add_body(r, carry):
                for u in range(2):
                    for d in range(HIDDEN // 16):
                        bufo[2 * r + u, pl.ds(d * 16, 16)] = (
                            ba[2 * r + u, pl.ds(d * 16, 16)]
                            + bb[2 * r + u, pl.ds(d * 16, 16)])
                return carry

            lax.fori_loop(0, CHUNK // 2, add_body, 0)
            pltpu.sync_copy(
                bufo, g_hbm.at[pl.ds(wbase + off, CHUNK), pl.ds(0, HIDDEN)])

        fire(0, bufa0, bufb0, sa0, sb0)

        def pair_body(j, carry):
            c0 = 2 * j
            fire(c0 + 1, bufa1, bufb1, sa1, sb1)
            drain(c0, bufa0, bufb0, sa0, sb0)
            fire(c0 + 2, bufa0, bufb0, sa0, sb0)
            drain(c0 + 1, bufa1, bufb1, sa1, sb1)
            return carry

        lax.fori_loop(0, (N_CHUNKS - 1) // 2, pair_body, 0)
        drain(N_CHUNKS - 1, bufa0, bufb0, sa0, sb0)

    return gather_add


BR = 6400


def _mlp_body(oinit_ref, eat_ref, g_ref, w1e_ref, w2_ref, b2_ref, w3_ref,
              b3_ref, gm_ref, bt_ref, out_ref):
    del oinit_ref
    h1 = lax.dot_general(eat_ref[...], w1e_ref[...],
                         (((0,), (0,)), ((), ())),
                         preferred_element_type=jnp.float32)
    h1 = jnp.maximum(h1 + g_ref[:, :HIDDEN], 0.0)
    h2 = jnp.dot(h1, w2_ref[...], preferred_element_type=jnp.float32)
    h2 = jnp.maximum(h2 + b2_ref[...], 0.0)
    ot = lax.dot_general(w3_ref[...], h2,
                         (((0,), (1,)), ((), ())),
                         preferred_element_type=jnp.float32) + b3_ref[...]
    mean = jnp.mean(ot, axis=0, keepdims=True)
    c = ot - mean
    var = jnp.mean(c * c, axis=0, keepdims=True)
    out_ref[...] = c * lax.rsqrt(var + 1e-5) * gm_ref[...] + bt_ref[...]


def _mlp(o_init, ea_t, g, blk0, w1e, w2, b2_row, w3, b3_col, gamma_col,
         beta_col):
    n_blocks = N_HALF // BR
    full = lambda i: (0, 0)
    return pl.pallas_call(
        _mlp_body,
        grid=(n_blocks,),
        in_specs=[
            pl.BlockSpec(memory_space=pl.ANY),
            pl.BlockSpec((EDGE_DIM, BR), lambda i: (0, i + blk0)),
            pl.BlockSpec((BR, 2 * HIDDEN), lambda i: (i, 0)),
            pl.BlockSpec((EDGE_DIM, HIDDEN), full),
            pl.BlockSpec((HIDDEN, HIDDEN), full),
            pl.BlockSpec((1, HIDDEN), full),
            pl.BlockSpec((HIDDEN, EDGE_DIM), full),
            pl.BlockSpec((EDGE_DIM, 1), full),
            pl.BlockSpec((EDGE_DIM, 1), full),
            pl.BlockSpec((EDGE_DIM, 1), full),
        ],
        out_specs=pl.BlockSpec((EDGE_DIM, BR), lambda i: (0, i + blk0)),
        out_shape=jax.ShapeDtypeStruct((EDGE_DIM, N_EDGES), jnp.float32),
        input_output_aliases={0: 0},
    )(o_init, ea_t, g, w1e, w2, b2_row, w3, b3_col, gamma_col, beta_col)


def kernel(edge_attr, node_attr, edge_index, W1, b1, W2, b2, W3, b3,
           gamma, beta):
    w1e = W1[:EDGE_DIM]
    w_sr = jnp.concatenate(
        [W1[EDGE_DIM:EDGE_DIM + NODE_DIM], W1[EDGE_DIM + NODE_DIM:]], axis=1)
    bias_row = jnp.concatenate(
        [jnp.zeros((HIDDEN,), jnp.float32), b1]).reshape(1, 2 * HIDDEN)
    p, q = _project_nodes(node_attr, w_sr, bias_row)
    row = edge_index[0]
    col = edge_index[1]
    ea_t = edge_attr.T
    b2r = b2.reshape(1, HIDDEN)
    b3c = b3.reshape(EDGE_DIM, 1)
    gmc = gamma.reshape(EDGE_DIM, 1)
    btc = beta.reshape(EDGE_DIM, 1)
    g0 = _make_gather_add(0)(p, q, row, col)
    g1 = _make_gather_add(N_HALF)(p, q, row, col)
    o_init = jnp.zeros((EDGE_DIM, N_EDGES), jnp.float32)
    o0 = _mlp(o_init, ea_t, g0, 0, w1e, W2, b2r, W3, b3c, gmc, btc)
    o1 = _mlp(o0, ea_t, g1, N_HALF // BR, w1e, W2, b2r, W3, b3c, gmc, btc)
    return o1.T


# speedup vs baseline: 1.3619x; 1.0820x over previous
import functools

import jax
import jax.numpy as jnp
from jax import lax
from jax.experimental import pallas as pl
from jax.experimental.pallas import tpu as pltpu
from jax.experimental.pallas import tpu_sc as plsc

N_NODES = 10000
N_EDGES = 320000
NODE_DIM = 128
EDGE_DIM = 16
HIDDEN = 64

NUM_CORES = 2
NUM_SUBCORES = 16
NW = NUM_CORES * NUM_SUBCORES
N_HALF = N_EDGES // 2
EPW = N_HALF // NW
CHUNK = 200
N_CHUNKS = EPW // CHUNK


def _proj_body(na_ref, w_ref, b1_ref, p_ref, q_ref):
    pq = jnp.dot(na_ref[...], w_ref[...], preferred_element_type=jnp.float32)
    t = pq + b1_ref[...]
    p_ref[...] = t[:, :HIDDEN]
    q_ref[...] = t[:, HIDDEN:]


def _project_nodes(node_attr, w_sr, bias_row):
    return pl.pallas_call(
        _proj_body,
        out_shape=(
            jax.ShapeDtypeStruct((N_NODES, HIDDEN), jnp.float32),
            jax.ShapeDtypeStruct((N_NODES, HIDDEN), jnp.float32),
        ),
    )(node_attr, w_sr, bias_row)


def _make_gather_add(e0):
    mesh = plsc.VectorSubcoreMesh(core_axis_name="c", subcore_axis_name="s")

    @functools.partial(
        pl.kernel,
        mesh=mesh,
        out_type=jax.ShapeDtypeStruct((N_HALF, 2 * HIDDEN), jnp.float32),
        compiler_params=pltpu.CompilerParams(use_tc_tiling_on_sc=False),
        scratch_types=[
            pltpu.VMEM((EPW,), jnp.int32),
            pltpu.VMEM((EPW,), jnp.int32),
            pltpu.VMEM((CHUNK, HIDDEN), jnp.float32),
            pltpu.VMEM((CHUNK, HIDDEN), jnp.float32),
            pltpu.VMEM((CHUNK, HIDDEN), jnp.float32),
            pltpu.VMEM((CHUNK, HIDDEN), jnp.float32),
            pltpu.VMEM((CHUNK, HIDDEN), jnp.float32),
            pltpu.SemaphoreType.DMA,
            pltpu.SemaphoreType.DMA,
            pltpu.SemaphoreType.DMA,
            pltpu.SemaphoreType.DMA,
        ],
    )
    def gather_add(p_hbm, q_hbm, ei_hbm, g_hbm,
                   rows, cols, bufa0, bufb0, bufa1, bufb1, bufo,
                   sa0, sb0, sa1, sb1):
        wid = lax.axis_index("s") * NUM_CORES + lax.axis_index("c")
        wbase = wid * EPW
        pltpu.sync_copy(ei_hbm.at[0, pl.ds(e0 + wbase, EPW)], rows)
        pltpu.sync_copy(ei_hbm.at[1, pl.ds(e0 + wbase, EPW)], cols)

        def fire(ci, ba, bb, sa, sb):
            off = ci * CHUNK
            pltpu.make_async_copy(
                p_hbm.at[rows.at[pl.ds(off, CHUNK)]], ba, sa).start()
            pltpu.make_async_copy(
                q_hbm.at[cols.at[pl.ds(off, CHUNK)]], bb, sb).start()

        def drain(ci, ba, bb, sa, sb):
            off = ci * CHUNK
            pltpu.make_async_copy(
                p_hbm.at[rows.at[pl.ds(off, CHUNK)]], ba, sa).wait()
            pltpu.make_async_copy(
                q_hbm.at[cols.at[pl.ds(off, CHUNK)]], bb, sb).wait()

            def add_body(r, carry):
                for u in range(2):
                    for d in range(HIDDEN // 16):
                        bufo[2 * r + u, pl.ds(d * 16, 16)] = (
                            ba[2 * r + u, pl.ds(d * 16, 16)]
                            + bb[2 * r + u, pl.ds(d * 16, 16)])
                return carry

            lax.fori_loop(0, CHUNK // 2, add_body, 0)
            pltpu.sync_copy(
                bufo, g_hbm.at[pl.ds(wbase + off, CHUNK), pl.ds(0, HIDDEN)])

        fire(0, bufa0, bufb0, sa0, sb0)

        def pair_body(j, carry):
            c0 = 2 * j
            fire(c0 + 1, bufa1, bufb1, sa1, sb1)
            drain(c0, bufa0, bufb0, sa0, sb0)
            fire(c0 + 2, bufa0, bufb0, sa0, sb0)
            drain(c0 + 1, bufa1, bufb1, sa1, sb1)
            return carry

        lax.fori_loop(0, (N_CHUNKS - 1) // 2, pair_body, 0)
        drain(N_CHUNKS - 1, bufa0, bufb0, sa0, sb0)

    return gather_add


BR = 6400


def _mlp_body(eat_ref, g_ref, w1e_ref, w2_ref, b2_ref, w3_ref,
              b3_ref, gm_ref, bt_ref, out_ref):
    h1 = lax.dot_general(eat_ref[...], w1e_ref[...],
                         (((0,), (0,)), ((), ())),
                         preferred_element_type=jnp.float32)
    h1 = jnp.maximum(h1 + g_ref[:, :HIDDEN], 0.0)
    h2 = jnp.dot(h1, w2_ref[...], preferred_element_type=jnp.float32)
    h2 = jnp.maximum(h2 + b2_ref[...], 0.0)
    ot = lax.dot_general(w3_ref[...], h2,
                         (((0,), (1,)), ((), ())),
                         preferred_element_type=jnp.float32) + b3_ref[...]
    mean = jnp.mean(ot, axis=0, keepdims=True)
    c = ot - mean
    var = jnp.mean(c * c, axis=0, keepdims=True)
    out_ref[...] = c * lax.rsqrt(var + 1e-5) * gm_ref[...] + bt_ref[...]


def _mlp(o_init, ea_t, g, blk0, w1e, w2, b2_row, w3, b3_col, gamma_col,
         beta_col):
    n_blocks = N_HALF // BR
    full = lambda i: (0, 0)
    init_spec = [] if o_init is None else [pl.BlockSpec(memory_space=pl.ANY)]
    init_arg = () if o_init is None else (o_init,)
    body = _mlp_body if o_init is None else (
        lambda oinit_ref, *refs: _mlp_body(*refs))
    return pl.pallas_call(
        body,
        grid=(n_blocks,),
        in_specs=init_spec + [
            pl.BlockSpec((EDGE_DIM, BR), lambda i: (0, i + blk0)),
            pl.BlockSpec((BR, 2 * HIDDEN), lambda i: (i, 0)),
            pl.BlockSpec((EDGE_DIM, HIDDEN), full),
            pl.BlockSpec((HIDDEN, HIDDEN), full),
            pl.BlockSpec((1, HIDDEN), full),
            pl.BlockSpec((HIDDEN, EDGE_DIM), full),
            pl.BlockSpec((EDGE_DIM, 1), full),
            pl.BlockSpec((EDGE_DIM, 1), full),
            pl.BlockSpec((EDGE_DIM, 1), full),
        ],
        out_specs=pl.BlockSpec((EDGE_DIM, BR), lambda i: (0, i + blk0)),
        out_shape=jax.ShapeDtypeStruct((EDGE_DIM, N_EDGES), jnp.float32),
        input_output_aliases={} if o_init is None else {0: 0},
    )(*init_arg, ea_t, g, w1e, w2, b2_row, w3, b3_col, gamma_col, beta_col)


def kernel(edge_attr, node_attr, edge_index, W1, b1, W2, b2, W3, b3,
           gamma, beta):
    w1e = W1[:EDGE_DIM]
    w_sr = jnp.concatenate(
        [W1[EDGE_DIM:EDGE_DIM + NODE_DIM], W1[EDGE_DIM + NODE_DIM:]], axis=1)
    bias_row = jnp.concatenate(
        [jnp.zeros((HIDDEN,), jnp.float32), b1]).reshape(1, 2 * HIDDEN)
    p, q = _project_nodes(node_attr, w_sr, bias_row)
    ea_t = edge_attr.T
    b2r = b2.reshape(1, HIDDEN)
    b3c = b3.reshape(EDGE_DIM, 1)
    gmc = gamma.reshape(EDGE_DIM, 1)
    btc = beta.reshape(EDGE_DIM, 1)
    g0 = _make_gather_add(0)(p, q, edge_index)
    g1 = _make_gather_add(N_HALF)(p, q, edge_index)
    o0 = _mlp(None, ea_t, g0, 0, w1e, W2, b2r, W3, b3c, gmc, btc)
    o1 = _mlp(o0, ea_t, g1, N_HALF // BR, w1e, W2, b2r, W3, b3c, gmc, btc)
    return o1.T
